# baseline (speedup 1.0000x reference)
import functools
import math

import jax
import jax.numpy as jnp
from jax import lax
from jax.experimental import pallas as pl
from jax.experimental.pallas import tpu as pltpu
from jax.experimental.pallas import tpu_sc as plsc

VOCAB = 1000000
D = 64
BATCH = 4096
SEQ = 200
B = BATCH * SEQ
NC, NS = 2, 16
NW = NC * NS
KS = SEQ // 8
KB = BATCH // 128
NUNITS = KS * KB * 4
U_PER_W = NUNITS // NW
B_PER_W = B // NW
NBUF = 4
NOB = 2
NOUT = U_PER_W // NBUF
SCALE = math.sqrt(D)


def _emb_call(x_flat, table):
    mesh = plsc.VectorSubcoreMesh(core_axis_name="c", subcore_axis_name="s")

    @functools.partial(
        pl.kernel,
        mesh=mesh,
        out_type=jax.ShapeDtypeStruct((SEQ, 8, KB, 1024), jnp.float32),
        compiler_params=pltpu.CompilerParams(
            use_tc_tiling_on_sc=False, needs_layout_passes=False
        ),
        scratch_types=[pltpu.VMEM((B_PER_W,), jnp.int32)]
        + [pltpu.VMEM((256, D), jnp.float32) for _ in range(NBUF)]
        + [pltpu.VMEM((2 * 8192,), jnp.float32) for _ in range(NOB)]
        + [pltpu.SemaphoreType.DMA] * NBUF
        + [pltpu.SemaphoreType.DMA] * NOB,
    )
    def emb_kernel(idx_hbm, table_hbm, out_hbm, idx_v, *rest):
        ibuf = rest[:NBUF]
        obuf = rest[NBUF:NBUF + NOB]
        gsem = rest[NBUF + NOB:NBUF + NOB + NBUF]
        osem = rest[NBUF + NOB + NBUF:]
        wid = lax.axis_index("s") * NC + lax.axis_index("c")
        ubase = wid * U_PER_W
        pltpu.sync_copy(idx_hbm.at[pl.ds(wid * B_PER_W, B_PER_W)], idx_v)

        def gathers(u, slot):
            return [
                pltpu.make_async_copy(
                    table_hbm.at[idx_v.at[pl.ds(u * 256 + h * 128, 128)]],
                    ibuf[slot].at[pl.ds(h * 128, 128)],
                    gsem[slot],
                )
                for h in range(2)
            ]

        def out_copies(u, oslot):
            ug = ubase + u
            ks = ug // 128
            rem = ug - ks * 128
            kb = rem // 4
            q = rem - kb * 4
            s0 = ks * 8 + 2 * q
            return [
                pltpu.make_async_copy(
                    obuf[oslot].at[pl.ds((j * 8 + kd) * 1024, 1024)],
                    out_hbm.at[s0 + j, kd, kb],
                    osem[oslot],
                )
                for j in range(2)
                for kd in range(8)
            ]

        def transpose_scale(slot, oslot):
            viota128 = lax.iota(jnp.int32, 16) * 128

            def gbody(g, c):
                idx_base = viota128 + g * 4
                for gi in range(4):
                    for j in range(2):
                        for cc in range(4):
                            v = ibuf[slot][
                                g * 4 + j * 128 + gi, pl.ds(cc * 16, 16)
                            ]
                            plsc.store_scatter(
                                obuf[oslot],
                                [idx_base + (j * 8192 + cc * 2048 + gi)],
                                v * SCALE,
                            )
                return c

            lax.fori_loop(0, 32, gbody, 0)

        for b in range(NBUF):
            for c in gathers(b, b):
                c.start()
        for b in range(NBUF):
            for c in gathers(b, b):
                c.wait()
            if b >= NOB:
                for c in out_copies(b - NOB, b % NOB):
                    c.wait()
            transpose_scale(b, b % NOB)
            for c in out_copies(b, b % NOB):
                c.start()
            for c in gathers(b + NBUF, b):
                c.start()

        def outer(o, carry):
            for b in range(NBUF):
                u = o * NBUF + b
                for c in gathers(u, b):
                    c.wait()
                for c in out_copies(u - NOB, b % NOB):
                    c.wait()
                transpose_scale(b, b % NOB)
                for c in out_copies(u, b % NOB):
                    c.start()
                for c in gathers(u + NBUF, b):
                    c.start()
            return carry

        lax.fori_loop(1, NOUT - 1, outer, 0)

        for b in range(NBUF):
            u = (NOUT - 1) * NBUF + b
            for c in gathers(u, b):
                c.wait()
            for c in out_copies(u - NOB, b % NOB):
                c.wait()
            transpose_scale(b, b % NOB)
            for c in out_copies(u, b % NOB):
                c.start()
        for b in range(NBUF - NOB, NBUF):
            u = (NOUT - 1) * NBUF + b
            for c in out_copies(u, b % NOB):
                c.wait()

    return emb_kernel(x_flat, table)


@jax.jit
def kernel(x, table):
    x4d = x.astype(jnp.int32).reshape(KB, 128, KS, 8).transpose(2, 0, 3, 1)
    out4d = _emb_call(x4d.reshape(-1), table)
    out5d = out4d.reshape(SEQ, 8, KB, 8, 128)
    return out5d.transpose(2, 4, 0, 1, 3).reshape(BATCH, SEQ, D)


# speedup vs baseline: 1.2898x; 1.2898x over previous
import functools
import math

import jax
import jax.numpy as jnp
from jax import lax
from jax.experimental import pallas as pl
from jax.experimental.pallas import tpu as pltpu
from jax.experimental.pallas import tpu_sc as plsc

VOCAB = 1000000
D = 64
BATCH = 4096
SEQ = 200
B = BATCH * SEQ
NC, NS = 2, 16
NW = NC * NS
KS = SEQ // 8
KB = BATCH // 128
NUNITS = KS * KB * 4
U_PER_W = NUNITS // NW
B_PER_W = B // NW
NBUF = 4
NOB = 2
NOUT = U_PER_W // NBUF
SCALE = math.sqrt(D)


def _emb_call(x_flat, table):
    mesh = plsc.VectorSubcoreMesh(core_axis_name="c", subcore_axis_name="s")

    @functools.partial(
        pl.kernel,
        mesh=mesh,
        out_type=jax.ShapeDtypeStruct((SEQ, 8, KB, 1024), jnp.float32),
        compiler_params=pltpu.CompilerParams(
            use_tc_tiling_on_sc=False, needs_layout_passes=False
        ),
        scratch_types=[pltpu.VMEM((B_PER_W,), jnp.int32)]
        + [pltpu.VMEM((256, D), jnp.float32) for _ in range(NBUF)]
        + [pltpu.VMEM((2 * 8192,), jnp.float32) for _ in range(NOB)]
        + [pltpu.SemaphoreType.DMA] * NBUF
        + [pltpu.SemaphoreType.DMA] * NOB,
    )
    def emb_kernel(idx_hbm, table_hbm, out_hbm, idx_v, *rest):
        ibuf = rest[:NBUF]
        obuf = rest[NBUF:NBUF + NOB]
        gsem = rest[NBUF + NOB:NBUF + NOB + NBUF]
        osem = rest[NBUF + NOB + NBUF:]
        wid = lax.axis_index("s") * NC + lax.axis_index("c")
        ubase = wid * U_PER_W
        pltpu.sync_copy(idx_hbm.at[pl.ds(wid * B_PER_W, B_PER_W)], idx_v)

        def gathers(u, slot):
            return [
                pltpu.make_async_copy(
                    table_hbm.at[idx_v.at[pl.ds(u * 256 + h * 128, 128)]],
                    ibuf[slot].at[pl.ds(h * 128, 128)],
                    gsem[slot],
                )
                for h in range(2)
            ]

        def out_copies(u, oslot):
            ug = ubase + u
            ks = ug // 128
            rem = ug - ks * 128
            kb = rem // 4
            q = rem - kb * 4
            s0 = ks * 8 + 2 * q
            return [
                pltpu.make_async_copy(
                    obuf[oslot].at[pl.ds((j * 8 + kd) * 1024, 1024)],
                    out_hbm.at[s0 + j, kd, kb],
                    osem[oslot],
                )
                for j in range(2)
                for kd in range(8)
            ]

        def transpose_scale(slot, oslot):
            viota128 = lax.iota(jnp.int32, 16) * 128

            @plsc.parallel_loop(0, 32, step=1, unroll=1)
            def gbody(g):
                idx_base = viota128 + g * 4
                for gi in range(4):
                    for j in range(2):
                        for cc in range(4):
                            v = ibuf[slot][
                                g * 4 + j * 128 + gi, pl.ds(cc * 16, 16)
                            ]
                            plsc.store_scatter(
                                obuf[oslot],
                                [idx_base + (j * 8192 + cc * 2048 + gi)],
                                v * SCALE,
                            )

        for b in range(NBUF):
            for c in gathers(b, b):
                c.start()
        for b in range(NBUF):
            for c in gathers(b, b):
                c.wait()
            if b >= NOB:
                for c in out_copies(b - NOB, b % NOB):
                    c.wait()
            transpose_scale(b, b % NOB)
            for c in out_copies(b, b % NOB):
                c.start()
            for c in gathers(b + NBUF, b):
                c.start()

        def outer(o, carry):
            for b in range(NBUF):
                u = o * NBUF + b
                for c in gathers(u, b):
                    c.wait()
                for c in out_copies(u - NOB, b % NOB):
                    c.wait()
                transpose_scale(b, b % NOB)
                for c in out_copies(u, b % NOB):
                    c.start()
                for c in gathers(u + NBUF, b):
                    c.start()
            return carry

        lax.fori_loop(1, NOUT - 1, outer, 0)

        for b in range(NBUF):
            u = (NOUT - 1) * NBUF + b
            for c in gathers(u, b):
                c.wait()
            for c in out_copies(u - NOB, b % NOB):
                c.wait()
            transpose_scale(b, b % NOB)
            for c in out_copies(u, b % NOB):
                c.start()
        for b in range(NBUF - NOB, NBUF):
            u = (NOUT - 1) * NBUF + b
            for c in out_copies(u, b % NOB):
                c.wait()

    return emb_kernel(x_flat, table)


@jax.jit
def kernel(x, table):
    x4d = x.astype(jnp.int32).reshape(KB, 128, KS, 8).transpose(2, 0, 3, 1)
    out4d = _emb_call(x4d.reshape(-1), table)
    out5d = out4d.reshape(SEQ, 8, KB, 8, 128)
    return out5d.transpose(2, 4, 0, 1, 3).reshape(BATCH, SEQ, D)
